# baseline (speedup 1.0000x reference)
import functools

import jax
import jax.numpy as jnp
from jax import lax
from jax.experimental import pallas as pl
from jax.experimental.pallas import tpu as pltpu

_TINY = 1.1754943508222875e-38
_ONE_BITS = 0x3F800000
_KS1 = 42
_KS2 = 0x1BD11BDA ^ 42
_ROT_A = (13, 15, 26, 6)
_ROT_B = (17, 29, 16, 24)

_CHUNK = 512
_WIDTH = 131072


def _rotl(v, r):
    return lax.shift_left(v, jnp.int32(r)) | lax.shift_right_logical(
        v, jnp.int32(32 - r))


def _four_rounds(x0, x1, rots):
    for r in rots:
        x0 = x0 + x1
        x1 = x0 ^ _rotl(x1, r)
    return x0, x1


def _threefry_bits(x1):
    ks1 = jnp.int32(_KS1)
    ks2 = jnp.int32(_KS2)
    x0 = x1
    x1 = x0 ^ _rotl(x1, _ROT_A[0])
    for r in _ROT_A[1:]:
        x0 = x0 + x1
        x1 = x0 ^ _rotl(x1, r)
    x0 = x0 + ks1
    x1 = x1 + (ks2 + jnp.int32(1))
    x0, x1 = _four_rounds(x0, x1, _ROT_B)
    x0 = x0 + ks2
    x1 = x1 + jnp.int32(2)
    x0, x1 = _four_rounds(x0, x1, _ROT_A)
    x1 = x1 + (ks1 + jnp.int32(3))
    x0, x1 = _four_rounds(x0, x1, _ROT_B)
    x0 = x0 + ks1
    x1 = x1 + (ks2 + jnp.int32(4))
    x0, x1 = _four_rounds(x0, x1, _ROT_A)
    x0 = x0 + ks2
    x1 = x1 + jnp.int32(5)
    return x0 ^ x1


def _gumbel_from_bits(bits):
    float_bits = lax.shift_right_logical(bits, jnp.int32(9)) | jnp.int32(
        _ONE_BITS)
    f = lax.bitcast_convert_type(float_bits, jnp.float32) - jnp.float32(1.0)
    u = f + jnp.float32(_TINY)
    return -jnp.log(-jnp.log(u))


def _chunk_scan(x_ref, rows, ncols, col_base, nch, valid_cols):
    best_v = jnp.full((rows, _CHUNK), -jnp.inf, jnp.float32)
    best_s = jnp.zeros((rows, _CHUNK), jnp.int32)
    col0 = lax.broadcasted_iota(jnp.int32, (rows, _CHUNK), 1)
    row_term = lax.broadcasted_iota(jnp.int32, (rows, _CHUNK), 0) * ncols
    ctr0 = row_term + col0 + jnp.int32(_KS1)
    for j in range(nch):
        off = j * _CHUNK
        xb = x_ref[:, off:off + _CHUNK]
        v = xb + _gumbel_from_bits(_threefry_bits(ctr0 + (col_base + off)))
        if valid_cols is not None and off + _CHUNK > valid_cols:
            v = jnp.where(col0 + off < valid_cols, v, -jnp.inf)
        upd = v > best_v
        best_v = jnp.where(upd, v, best_v)
        best_s = jnp.where(upd, jnp.int32((col_base + off) // _CHUNK),
                           best_s)
    return best_v, best_s


def _lane_reduce(best_v, best_s):
    col0 = lax.broadcasted_iota(jnp.int32, (best_v.shape[0], _CHUNK), 1)
    best_c = best_s * jnp.int32(_CHUNK) + col0
    m = jnp.max(best_v, axis=1, keepdims=True)
    idx = jnp.min(
        jnp.where(best_v == m, best_c, jnp.int32(0x7FFFFFFF)),
        axis=1,
        keepdims=True)
    return m, idx


def _tail_body(x_ref, outv_ref, outc_ref, *, rows, ncols, col_base,
               valid_cols, nch):
    best_v, best_c = _chunk_scan(x_ref, rows, ncols, col_base, nch,
                                 valid_cols)
    m, idx = _lane_reduce(best_v, best_c)
    outv_ref[...] = m
    outc_ref[...] = idx


def _main_body(x_ref, tv_ref, tc_ref, out_ref, bv_ref, bc_ref, *, rows,
               ncols, width, nblk):
    b = pl.program_id(0)
    best_v, best_c = _chunk_scan(x_ref, rows, ncols, b * width,
                                 width // _CHUNK, None)

    @pl.when(b == 0)
    def _init():
        bv_ref[...] = best_v
        bc_ref[...] = best_c

    @pl.when(b > 0)
    def _merge():
        upd = best_v > bv_ref[...]
        bv_ref[...] = jnp.where(upd, best_v, bv_ref[...])
        bc_ref[...] = jnp.where(upd, best_c, bc_ref[...])

    @pl.when(b == nblk - 1)
    def _fin():
        m, idx = _lane_reduce(bv_ref[...], bc_ref[...])
        idx = jnp.where(tv_ref[...] > m, tc_ref[...], idx)
        out_ref[...] = idx


def _round_up(n, m):
    return (n + m - 1) // m * m


@functools.partial(jax.jit, static_argnames=())
def kernel(x):
    rows, ncols = x.shape
    width = _WIDTH
    nfull = ncols // width
    if nfull == 0:
        width = (ncols // _CHUNK) * _CHUNK
        nfull = 1
    main_cols = nfull * width
    tail_len = ncols - main_cols

    if tail_len > 0:
        tail_pad = _round_up(tail_len, _CHUNK)
        xt = jnp.pad(
            lax.slice(x, (0, main_cols), (rows, ncols)),
            ((0, 0), (0, tail_pad - tail_len)))
        tv, tc = pl.pallas_call(
            functools.partial(
                _tail_body,
                rows=rows,
                ncols=ncols,
                col_base=main_cols,
                valid_cols=tail_len,
                nch=tail_pad // _CHUNK),
            out_shape=(
                jax.ShapeDtypeStruct((rows, 1), jnp.float32),
                jax.ShapeDtypeStruct((rows, 1), jnp.int32),
            ),
        )(xt)
    else:
        tv = jnp.full((rows, 1), -jnp.inf, jnp.float32)
        tc = jnp.zeros((rows, 1), jnp.int32)

    out = pl.pallas_call(
        functools.partial(
            _main_body, rows=rows, ncols=ncols, width=width, nblk=nfull),
        grid=(nfull,),
        in_specs=[
            pl.BlockSpec((rows, width), lambda b: (0, b)),
            pl.BlockSpec((rows, 1), lambda b: (0, 0)),
            pl.BlockSpec((rows, 1), lambda b: (0, 0)),
        ],
        out_specs=pl.BlockSpec((rows, 1), lambda b: (0, 0)),
        out_shape=jax.ShapeDtypeStruct((rows, 1), jnp.int32),
        scratch_shapes=[
            pltpu.VMEM((rows, _CHUNK), jnp.float32),
            pltpu.VMEM((rows, _CHUNK), jnp.int32),
        ],
    )(x, tv, tc)
    return out.reshape(rows)


# speedup vs baseline: 1.1095x; 1.1095x over previous
import functools

import jax
import jax.numpy as jnp
from jax import lax
from jax.experimental import pallas as pl
from jax.experimental.pallas import tpu as pltpu

_TINY = 1.1754943508222875e-38
_ONE_BITS = 0x3F800000
_KS1 = 42
_KS2 = 0x1BD11BDA ^ 42
_ROT_A = (13, 15, 26, 6)
_ROT_B = (17, 29, 16, 24)

_CHUNK = 512
_WIDTH = 65536


def _rotl(v, r):
    return lax.shift_left(v, jnp.int32(r)) | lax.shift_right_logical(
        v, jnp.int32(32 - r))


def _four_rounds(x0, x1, rots):
    for r in rots:
        x0 = x0 + x1
        x1 = x0 ^ _rotl(x1, r)
    return x0, x1


def _threefry_bits(x1):
    ks1 = jnp.int32(_KS1)
    ks2 = jnp.int32(_KS2)
    x0 = x1
    x1 = x0 ^ _rotl(x1, _ROT_A[0])
    for r in _ROT_A[1:]:
        x0 = x0 + x1
        x1 = x0 ^ _rotl(x1, r)
    x0 = x0 + ks1
    x1 = x1 + (ks2 + jnp.int32(1))
    x0, x1 = _four_rounds(x0, x1, _ROT_B)
    x0 = x0 + ks2
    x1 = x1 + jnp.int32(2)
    x0, x1 = _four_rounds(x0, x1, _ROT_A)
    x1 = x1 + (ks1 + jnp.int32(3))
    x0, x1 = _four_rounds(x0, x1, _ROT_B)
    x0 = x0 + ks1
    x1 = x1 + (ks2 + jnp.int32(4))
    x0, x1 = _four_rounds(x0, x1, _ROT_A)
    x0 = x0 + ks2
    x1 = x1 + jnp.int32(5)
    return x0 ^ x1


def _gumbel_from_bits(bits):
    float_bits = lax.shift_right_logical(bits, jnp.int32(9)) | jnp.int32(
        _ONE_BITS)
    f = lax.bitcast_convert_type(float_bits, jnp.float32) - jnp.float32(1.0)
    u = f + jnp.float32(_TINY)
    return -jnp.log(-jnp.log(u))


def _chunk_scan(x_ref, rows, ncols, col_base, nch, valid_cols):
    best_v = jnp.full((rows, _CHUNK), -jnp.inf, jnp.float32)
    best_s = jnp.zeros((rows, _CHUNK), jnp.int32)
    col0 = lax.broadcasted_iota(jnp.int32, (rows, _CHUNK), 1)
    row_term = lax.broadcasted_iota(jnp.int32, (rows, _CHUNK), 0) * ncols
    ctr0 = row_term + col0 + jnp.int32(_KS1)
    for j in range(nch):
        off = j * _CHUNK
        xb = x_ref[:, off:off + _CHUNK]
        v = xb + _gumbel_from_bits(_threefry_bits(ctr0 + (col_base + off)))
        if valid_cols is not None and off + _CHUNK > valid_cols:
            v = jnp.where(col0 + off < valid_cols, v, -jnp.inf)
        upd = v > best_v
        best_v = jnp.where(upd, v, best_v)
        best_s = jnp.where(upd, jnp.int32((col_base + off) // _CHUNK),
                           best_s)
    return best_v, best_s


def _lane_reduce(best_v, best_s):
    col0 = lax.broadcasted_iota(jnp.int32, (best_v.shape[0], _CHUNK), 1)
    best_c = best_s * jnp.int32(_CHUNK) + col0
    m = jnp.max(best_v, axis=1, keepdims=True)
    idx = jnp.min(
        jnp.where(best_v == m, best_c, jnp.int32(0x7FFFFFFF)),
        axis=1,
        keepdims=True)
    return m, idx


def _tail_body(x_ref, outv_ref, outc_ref, *, rows, ncols, col_base,
               valid_cols, nch):
    best_v, best_c = _chunk_scan(x_ref, rows, ncols, col_base, nch,
                                 valid_cols)
    m, idx = _lane_reduce(best_v, best_c)
    outv_ref[...] = m
    outc_ref[...] = idx


def _main_body(x_ref, tv_ref, tc_ref, out_ref, bv_ref, bc_ref, *, rows,
               ncols, width, nblk):
    b = pl.program_id(0)
    best_v, best_c = _chunk_scan(x_ref, rows, ncols, b * width,
                                 width // _CHUNK, None)

    @pl.when(b == 0)
    def _init():
        bv_ref[...] = best_v
        bc_ref[...] = best_c

    @pl.when(b > 0)
    def _merge():
        upd = best_v > bv_ref[...]
        bv_ref[...] = jnp.where(upd, best_v, bv_ref[...])
        bc_ref[...] = jnp.where(upd, best_c, bc_ref[...])

    @pl.when(b == nblk - 1)
    def _fin():
        m, idx = _lane_reduce(bv_ref[...], bc_ref[...])
        idx = jnp.where(tv_ref[...] > m, tc_ref[...], idx)
        out_ref[...] = idx


def _round_up(n, m):
    return (n + m - 1) // m * m


@functools.partial(jax.jit, static_argnames=())
def kernel(x):
    rows, ncols = x.shape
    width = _WIDTH
    nfull = ncols // width
    if nfull == 0:
        width = (ncols // _CHUNK) * _CHUNK
        nfull = 1
    main_cols = nfull * width
    tail_len = ncols - main_cols

    if tail_len > 0:
        tail_pad = _round_up(tail_len, _CHUNK)
        xt = jnp.pad(
            lax.slice(x, (0, main_cols), (rows, ncols)),
            ((0, 0), (0, tail_pad - tail_len)))
        tv, tc = pl.pallas_call(
            functools.partial(
                _tail_body,
                rows=rows,
                ncols=ncols,
                col_base=main_cols,
                valid_cols=tail_len,
                nch=tail_pad // _CHUNK),
            out_shape=(
                jax.ShapeDtypeStruct((rows, 1), jnp.float32),
                jax.ShapeDtypeStruct((rows, 1), jnp.int32),
            ),
        )(xt)
    else:
        tv = jnp.full((rows, 1), -jnp.inf, jnp.float32)
        tc = jnp.zeros((rows, 1), jnp.int32)

    out = pl.pallas_call(
        functools.partial(
            _main_body, rows=rows, ncols=ncols, width=width, nblk=nfull),
        grid=(nfull,),
        in_specs=[
            pl.BlockSpec((rows, width), lambda b: (0, b)),
            pl.BlockSpec((rows, 1), lambda b: (0, 0)),
            pl.BlockSpec((rows, 1), lambda b: (0, 0)),
        ],
        out_specs=pl.BlockSpec((rows, 1), lambda b: (0, 0)),
        out_shape=jax.ShapeDtypeStruct((rows, 1), jnp.int32),
        scratch_shapes=[
            pltpu.VMEM((rows, _CHUNK), jnp.float32),
            pltpu.VMEM((rows, _CHUNK), jnp.int32),
        ],
    )(x, tv, tc)
    return out.reshape(rows)
